# baseline (speedup 1.0000x reference)
import functools

import jax
import jax.numpy as jnp
from jax import lax
from jax.experimental import pallas as pl
from jax.experimental.pallas import tpu as pltpu
from jax.experimental.pallas import tpu_sc as plsc

NSC = 2
TPS = 16
K = 512
NRING = 4
LANES = 16


@functools.partial(jax.jit, static_argnums=(3, 4, 5))
def _sc_aggregate(x2, pk, pw, N, DH, NCH):
    rpt = 8 * (-(-N // (TPS * 8)))
    NP = TPS * rpt
    assert NCH % NRING == 0 and NCH >= NRING

    mesh = plsc.VectorSubcoreMesh(core_axis_name="c", subcore_axis_name="s")

    @functools.partial(
        pl.kernel,
        mesh=mesh,
        out_type=jax.ShapeDtypeStruct((NSC, NP, DH), jnp.float32),
        scratch_types=[
            pltpu.VMEM((NRING, 2, K), jnp.int32),
            pltpu.VMEM((NRING, K), jnp.float32),
            pltpu.VMEM((K, DH), jnp.float32),
            pltpu.VMEM((K, DH), jnp.float32),
            pltpu.VMEM_SHARED((NP, DH), jnp.float32),
        ]
        + [pltpu.SemaphoreType.DMA for _ in range(NRING + 4)],
        compiler_params=pltpu.CompilerParams(use_tc_tiling_on_sc=False),
    )
    def sc(x_hbm, pk_hbm, pw_hbm, out_hbm, idxr, ewr, buf0, buf1, acc,
           *sems):
        si = sems[:NRING]
        sg = sems[NRING:NRING + 2]
        ss = sems[NRING + 2:NRING + 4]
        bufs = (buf0, buf1)
        cid = lax.axis_index("c")
        sid = lax.axis_index("s")

        def zrow(r, carry):
            for c in range(DH // LANES):
                buf0[r, pl.ds(c * LANES, LANES)] = jnp.zeros(
                    (LANES,), jnp.float32)
            return carry
        lax.fori_loop(0, K, zrow, 0)

        zbase = sid * rpt
        nfull = rpt // K
        rem = rpt - nfull * K

        def zcp(i, carry):
            pltpu.sync_copy(buf0, acc.at[pl.ds(zbase + i * K, K)])
            return carry
        lax.fori_loop(0, nfull, zcp, 0)
        if rem:
            pltpu.sync_copy(buf0.at[pl.ds(0, rem)],
                            acc.at[pl.ds(zbase + nfull * K, rem)])

        def add_cid(slot):
            def grp(g, carry):
                sl = pl.ds(g * LANES, LANES)
                idxr[slot, 0, sl] = idxr[slot, 0, sl] + cid
                return carry
            lax.fori_loop(0, K // LANES, grp, 0)

        for s in range(NRING - 1):
            pltpu.async_copy(pk_hbm.at[sid, s], idxr.at[s], si[s])
            pltpu.async_copy(pw_hbm.at[sid, s], ewr.at[s], si[s])
        pltpu.make_async_copy(pk_hbm.at[sid, 0], idxr.at[0], si[0]).wait()
        pltpu.make_async_copy(pw_hbm.at[sid, 0], ewr.at[0], si[0]).wait()
        add_cid(0)
        pltpu.async_copy(x_hbm.at[idxr.at[0, 0]], bufs[0], sg[0])

        plsc.subcore_barrier()

        def scale(slot, buf):
            def rowblk(g, c2):
                wv = ewr[slot, pl.ds(g * LANES, LANES)]
                for u in range(LANES):
                    w = wv[u]
                    r = g * LANES + u
                    for c in range(DH // LANES):
                        sl = pl.ds(c * LANES, LANES)
                        buf[r, sl] = buf[r, sl] * w
                return c2
            lax.fori_loop(0, K // LANES, rowblk, 0)

        def quad(i, carry):
            for uu in range(NRING):
                j = i * NRING + uu
                u = uu % 2
                un = (uu + 1) % 2
                sn = (uu + 1) % NRING
                sf = (uu + 3) % NRING

                @pl.when(j + 1 < NCH)
                def _():
                    @pl.when(j >= 1)
                    def _():
                        pltpu.make_async_copy(
                            bufs[un], acc.at[idxr.at[sf, 1]], ss[un]).wait()
                    jp = jnp.minimum(j + 1, NCH - 1)
                    pltpu.make_async_copy(
                        pk_hbm.at[sid, jp], idxr.at[sn], si[sn]).wait()
                    pltpu.make_async_copy(
                        pw_hbm.at[sid, jp], ewr.at[sn], si[sn]).wait()
                    add_cid(sn)
                    pltpu.async_copy(
                        x_hbm.at[idxr.at[sn, 0]], bufs[un], sg[un])

                @pl.when(j + 3 < NCH)
                def _():
                    jf = jnp.minimum(j + 3, NCH - 1)
                    pltpu.async_copy(
                        pk_hbm.at[sid, jf], idxr.at[sf], si[sf])
                    pltpu.async_copy(
                        pw_hbm.at[sid, jf], ewr.at[sf], si[sf])

                pltpu.make_async_copy(
                    x_hbm.at[idxr.at[uu, 0]], bufs[u], sg[u]).wait()
                scale(uu, bufs[u])
                pltpu.async_copy(bufs[u], acc.at[idxr.at[uu, 1]], ss[u],
                                 add=True)
            return carry
        lax.fori_loop(0, NCH // NRING, quad, 0)

        for uu in range(NRING - 2, NRING):
            pltpu.make_async_copy(
                bufs[uu % 2], acc.at[idxr.at[uu, 1]], ss[uu % 2]).wait()

        plsc.subcore_barrier()
        pltpu.sync_copy(acc.at[pl.ds(zbase, rpt)],
                        out_hbm.at[cid, pl.ds(zbase, rpt)])

    return sc(x2, pk, pw)


def _tc_combine_matmul(P, W, b, N):
    _, _, DH = P.shape
    DO = W.shape[1]
    BM = 1000

    def body(p_ref, w_ref, b_ref, o_ref):
        o_ref[...] = (
            jnp.dot(p_ref[0], w_ref[0:DH, :],
                    preferred_element_type=jnp.float32)
            + jnp.dot(p_ref[1], w_ref[DH:2 * DH, :],
                      preferred_element_type=jnp.float32)
            + b_ref[...]
        )

    return pl.pallas_call(
        body,
        grid=(N // BM,),
        in_specs=[
            pl.BlockSpec((NSC, BM, DH), lambda i: (0, i, 0)),
            pl.BlockSpec((2 * DH, DO), lambda i: (0, 0)),
            pl.BlockSpec((1, DO), lambda i: (0, 0)),
        ],
        out_specs=pl.BlockSpec((BM, DO), lambda i: (i, 0)),
        out_shape=jax.ShapeDtypeStruct((N, DO), jnp.float32),
    )(P, W, b.reshape(1, DO))


def kernel(input, adj, edge_weight, W, b):
    x = input
    N, D = x.shape
    DH = D // NSC
    E = edge_weight.shape[0]

    ept = -(-E // TPS)
    NCH = NRING * (-(-ept // (NRING * K)))
    EPT = NCH * K

    col = adj[1]
    row = adj[0]
    if E % TPS:
        pad0 = TPS * ept - E
        col = jnp.pad(col, (0, pad0))
        row = jnp.pad(row, (0, pad0))
        ew = jnp.pad(edge_weight, (0, pad0))
    else:
        ew = edge_weight
    colv = jnp.pad((col * 2).reshape(TPS, ept), ((0, 0), (0, EPT - ept)))
    rowv = jnp.pad(row.reshape(TPS, ept), ((0, 0), (0, EPT - ept)))
    eww = jnp.pad(ew.reshape(TPS, ept), ((0, 0), (0, EPT - ept)))

    pk = jnp.stack(
        [colv.reshape(TPS, NCH, K), rowv.reshape(TPS, NCH, K)], axis=2)
    pw = eww.reshape(TPS, NCH, K)

    P = _sc_aggregate(x.reshape(NSC * N, DH), pk, pw, N, DH, NCH)
    return _tc_combine_matmul(P, W, b, N)


# speedup vs baseline: 1.8655x; 1.8655x over previous
import functools

import jax
import jax.numpy as jnp
from jax import lax
from jax.experimental import pallas as pl
from jax.experimental.pallas import tpu as pltpu
from jax.experimental.pallas import tpu_sc as plsc

NSC = 2
TPS = 16
NT = NSC * TPS
K = 128
LANES = 16


@functools.partial(jax.jit, static_argnums=(4, 5, 6))
def _sc_aggregate(x, colv, rowv, eww, N, D, NCH):
    rpt = 8 * (-(-N // (TPS * 8)))
    NP = TPS * rpt

    mesh = plsc.VectorSubcoreMesh(core_axis_name="c", subcore_axis_name="s")

    @functools.partial(
        pl.kernel,
        mesh=mesh,
        out_type=jax.ShapeDtypeStruct((NSC, NP, D), jnp.float32),
        scratch_types=[
            pltpu.VMEM((NCH, K), jnp.int32),
            pltpu.VMEM((NCH, K), jnp.int32),
            pltpu.VMEM((NCH, K), jnp.float32),
            pltpu.VMEM((K, D), jnp.float32),
            pltpu.VMEM_SHARED((NP, D), jnp.float32),
            pltpu.SemaphoreType.DMA,
        ],
    )
    def sc(x_hbm, col_hbm, row_hbm, ew_hbm, out_hbm,
           colr, rowr, ewr, rbuf, acc, sem):
        cid = lax.axis_index("c")
        sid = lax.axis_index("s")
        tid = cid * TPS + sid

        pltpu.sync_copy(col_hbm.at[tid], colr)
        pltpu.sync_copy(row_hbm.at[tid], rowr)
        pltpu.sync_copy(ew_hbm.at[tid], ewr)

        def zrow(r, carry):
            for c in range(D // LANES):
                rbuf[r, pl.ds(c * LANES, LANES)] = jnp.zeros(
                    (LANES,), jnp.float32)
            return carry
        lax.fori_loop(0, K, zrow, 0)

        zbase = sid * rpt
        nfull = rpt // K
        rem = rpt - nfull * K

        def zcp(i, carry):
            pltpu.sync_copy(rbuf, acc.at[pl.ds(zbase + i * K, K)])
            return carry
        lax.fori_loop(0, nfull, zcp, 0)
        if rem:
            pltpu.sync_copy(rbuf.at[pl.ds(0, rem)],
                            acc.at[pl.ds(zbase + nfull * K, rem)])
        plsc.subcore_barrier()

        def chunk(j, carry):
            pltpu.async_copy(x_hbm.at[colr.at[j]], rbuf, sem).wait()

            def rowblk(g, c2):
                wv = ewr[j, pl.ds(g * LANES, LANES)]
                for u in range(LANES):
                    w = wv[u]
                    r = g * LANES + u
                    for c in range(D // LANES):
                        sl = pl.ds(c * LANES, LANES)
                        rbuf[r, sl] = rbuf[r, sl] * w
                return c2
            lax.fori_loop(0, K // LANES, rowblk, 0)

            pltpu.sync_copy(rbuf, acc.at[rowr.at[j]], add=True)
            return carry
        lax.fori_loop(0, NCH, chunk, 0)

        plsc.subcore_barrier()
        pltpu.sync_copy(acc.at[pl.ds(zbase, rpt)],
                        out_hbm.at[cid, pl.ds(zbase, rpt)])

    return sc(x, colv, rowv, eww)


def _tc_combine_matmul(P, W, b, N):
    _, _, D = P.shape
    DO = W.shape[1]
    BM = 1000

    def body(p_ref, w_ref, b_ref, o_ref):
        s = p_ref[0] + p_ref[1]
        o_ref[...] = (
            jnp.dot(s, w_ref[...], preferred_element_type=jnp.float32)
            + b_ref[...]
        )

    return pl.pallas_call(
        body,
        grid=(N // BM,),
        in_specs=[
            pl.BlockSpec((NSC, BM, D), lambda i: (0, i, 0)),
            pl.BlockSpec((D, DO), lambda i: (0, 0)),
            pl.BlockSpec((1, DO), lambda i: (0, 0)),
        ],
        out_specs=pl.BlockSpec((BM, DO), lambda i: (i, 0)),
        out_shape=jax.ShapeDtypeStruct((N, DO), jnp.float32),
    )(P, W, b.reshape(1, DO))


def kernel(input, adj, edge_weight, W, b):
    x = input
    N, D = x.shape
    E = edge_weight.shape[0]

    ept = -(-E // NT)
    NCH = -(-ept // K)
    EPT = NCH * K

    col = adj[1]
    row = adj[0]
    if E % NT:
        pad0 = NT * ept - E
        col = jnp.pad(col, (0, pad0))
        row = jnp.pad(row, (0, pad0))
        ew = jnp.pad(edge_weight, (0, pad0))
    else:
        ew = edge_weight
    colv = jnp.pad(col.reshape(NT, ept), ((0, 0), (0, EPT - ept)))
    rowv = jnp.pad(row.reshape(NT, ept), ((0, 0), (0, EPT - ept)))
    eww = jnp.pad(ew.reshape(NT, ept), ((0, 0), (0, EPT - ept)))

    P = _sc_aggregate(
        x,
        colv.reshape(NT, NCH, K),
        rowv.reshape(NT, NCH, K),
        eww.reshape(NT, NCH, K),
        N, D, NCH,
    )
    return _tc_combine_matmul(P, W, b, N)
